# Initial kernel scaffold; baseline (speedup 1.0000x reference)
import functools

import jax
import jax.numpy as jnp
from jax import lax
from jax.experimental import pallas as pl
from jax.experimental.pallas import tpu as pltpu
from jax.experimental.pallas import tpu_sc as plsc

_N = 24256
_E = 388096
_D = 128
_B = 64
_R = 379
_RP = 384
_HF = 64
_NT = 16
_NC = 2
_BN = 3032

_f32 = jnp.float32
_i32 = jnp.int32


def _mesh():
    return plsc.VectorSubcoreMesh(core_axis_name="c", subcore_axis_name="s")


def _table_body(lew_ref, t_ref):
    a = lew_ref[...]
    t_ref[...] = 2.0 * jax.nn.sigmoid((a + a.T) * 0.5)


def _tc_table(lew_pad):
    return pl.pallas_call(
        _table_body,
        out_shape=jax.ShapeDtypeStruct((_RP, _RP), _f32),
    )(lew_pad)


def _deg_body(dp_ref, dinv_ref, dinv2_ref):
    dp = dp_ref[...]
    deg = dp[:8] + dp[8:] + 1.0
    dinv_ref[...] = lax.rsqrt(deg)
    dinv2_ref[...] = 1.0 / deg


def _tc_deg(degp):
    return pl.pallas_call(
        _deg_body,
        out_shape=[
            jax.ShapeDtypeStruct((8, _BN), _f32),
            jax.ShapeDtypeStruct((8, _BN), _f32),
        ],
    )(degp)


def _mm_body(x_ref, w1_ref, we_ref, be_ref, h1_ref, xe_ref):
    xb = x_ref[...]
    h1 = jnp.dot(xb, w1_ref[...], preferred_element_type=_f32)
    xe = jnp.dot(xb, we_ref[...], preferred_element_type=_f32) + be_ref[...]
    xe = jnp.maximum(xe, 0.0)
    h1_ref[0] = h1[:, :_HF]
    h1_ref[1] = h1[:, _HF:]
    xe_ref[0] = xe[:, :_HF]
    xe_ref[1] = xe[:, _HF:]


def _tc_mm(x, w1, we, be_row):
    grid = _N // _BN
    return pl.pallas_call(
        _mm_body,
        grid=(grid,),
        in_specs=[
            pl.BlockSpec((_BN, _D), lambda i: (i, 0)),
            pl.BlockSpec((_D, _D), lambda i: (0, 0)),
            pl.BlockSpec((_D, _D), lambda i: (0, 0)),
            pl.BlockSpec((1, _D), lambda i: (0, 0)),
        ],
        out_specs=[
            pl.BlockSpec((2, _BN, _HF), lambda i: (0, i, 0)),
            pl.BlockSpec((2, _BN, _HF), lambda i: (0, i, 0)),
        ],
        out_shape=[
            jax.ShapeDtypeStruct((2, _N, _HF), _f32),
            jax.ShapeDtypeStruct((2, _N, _HF), _f32),
        ],
    )(x, w1, we, be_row)


def _l1_body(agg_ref, h1_ref, xe_ref, d2_ref, b1_ref, w2_ref, x1_ref, h2_ref):
    d2 = d2_ref[...]
    b1 = b1_ref[...]
    x1h = []
    for t in range(2):
        o = agg_ref[t] + h1_ref[t] * d2 + b1[:, _HF * t:_HF * (t + 1)]
        x1h.append(jnp.maximum(o, 0.0) + xe_ref[t])
    w2 = w2_ref[...]
    h2 = (jnp.dot(x1h[0], w2[:_HF, :], preferred_element_type=_f32)
          + jnp.dot(x1h[1], w2[_HF:, :], preferred_element_type=_f32))
    x1_ref[0] = x1h[0]
    x1_ref[1] = x1h[1]
    h2_ref[0] = h2[:, :_HF]
    h2_ref[1] = h2[:, _HF:]


def _tc_l1(agg1, h1, xe, d2col, b1_row, w2):
    grid = _N // _BN
    half_spec = pl.BlockSpec((2, _BN, _HF), lambda i: (0, i, 0))
    return pl.pallas_call(
        _l1_body,
        grid=(grid,),
        in_specs=[
            half_spec,
            half_spec,
            half_spec,
            pl.BlockSpec((_BN, 1), lambda i: (i, 0)),
            pl.BlockSpec((1, _D), lambda i: (0, 0)),
            pl.BlockSpec((_D, _D), lambda i: (0, 0)),
        ],
        out_specs=[half_spec, half_spec],
        out_shape=[
            jax.ShapeDtypeStruct((2, _N, _HF), _f32),
            jax.ShapeDtypeStruct((2, _N, _HF), _f32),
        ],
    )(agg1, h1, xe, d2col, b1_row, w2)


def _l2_body(agg_ref, h2_ref, x1_ref, d2_ref, b2_ref, x2_ref):
    d2 = d2_ref[...]
    b2 = b2_ref[...]
    for t in range(2):
        o = agg_ref[t] + h2_ref[t] * d2 + b2[:, _HF * t:_HF * (t + 1)]
        x2_ref[t] = jnp.maximum(o, 0.0) + x1_ref[t]


def _tc_l2(agg2, h2, x1, d2col, b2_row):
    grid = _N // _BN
    half_spec = pl.BlockSpec((2, _BN, _HF), lambda i: (0, i, 0))
    return pl.pallas_call(
        _l2_body,
        grid=(grid,),
        in_specs=[
            half_spec,
            half_spec,
            half_spec,
            pl.BlockSpec((_BN, 1), lambda i: (i, 0)),
            pl.BlockSpec((1, _D), lambda i: (0, 0)),
        ],
        out_specs=half_spec,
        out_shape=jax.ShapeDtypeStruct((2, _N, _HF), _f32),
    )(agg2, h2, x1, d2col, b2_row)


def _pred_body(sums_ref, cnt_ref, wf_ref, bf_ref, out_ref):
    cnt = jnp.maximum(cnt_ref[...][:, 0:1], 1.0)
    wf = wf_ref[...]
    p0 = sums_ref[0] / cnt
    p1 = sums_ref[1] / cnt
    out_ref[...] = (jnp.dot(p0, wf[:_HF, :], preferred_element_type=_f32)
                    + jnp.dot(p1, wf[_HF:, :], preferred_element_type=_f32)
                    + bf_ref[...])


def _tc_pred(sums, cnt16, wf, bf_row):
    return pl.pallas_call(
        _pred_body,
        out_shape=jax.ShapeDtypeStruct((_B, 1), _f32),
    )(sums, cnt16, wf, bf_row)


_EW_TILE = _E // (_NC * _NT)
_EW_CH = 128
_EW_FULL = _EW_TILE // _EW_CH
_EW_TAIL = _EW_TILE - _EW_FULL * _EW_CH

_DEG_SL = _N // 8


def _wdeg_body(row_h, col_h, tflat_h, w_h, degp_h,
               rbuf, cbuf, fbuf, wstage, rtb, ctb, ftb, wtb, zb, degS):
    ci = lax.axis_index("c")
    s = lax.axis_index("s")
    wid = ci * _NT + s
    tbase = wid * _EW_TILE

    @pl.loop(0, 192)
    def _z(i):
        zb[pl.ds(i * 16, 16)] = jnp.zeros((16,), _f32)

    @pl.when(s < 8)
    def _zdeg():
        pltpu.sync_copy(zb.at[pl.ds(0, _DEG_SL)],
                        degS.at[pl.ds(s * _DEG_SL, _DEG_SL)])

    plsc.subcore_barrier()

    def _chunk(base, ch, rb, cb, fb, wst):
        pltpu.sync_copy(row_h.at[pl.ds(base, ch)], rb)
        pltpu.sync_copy(col_h.at[pl.ds(base, ch)], cb)
        for g in range(ch // 16):
            sl = pl.ds(g * 16, 16)
            rv = rb[sl]
            cv = cb[sl]
            fb[sl] = (rv % _R) * _RP + (cv % _R)
        pltpu.sync_copy(tflat_h.at[fb], wst)
        pltpu.sync_copy(wst, w_h.at[pl.ds(base, ch)])
        pltpu.sync_copy(wst, degS.at[cb], add=True)

    @pl.loop(0, _EW_FULL)
    def _c(i):
        _chunk(tbase + i * _EW_CH, _EW_CH, rbuf, cbuf, fbuf, wstage)

    _chunk(tbase + _EW_FULL * _EW_CH, _EW_TAIL, rtb, ctb, ftb, wtb)

    plsc.subcore_barrier()

    @pl.when(s < 8)
    def _out():
        pltpu.sync_copy(degS.at[pl.ds(s * _DEG_SL, _DEG_SL)],
                        degp_h.at[pl.ds(ci * _N + s * _DEG_SL, _DEG_SL)])


def _sc_wdeg(row, col, tflat):
    k = pl.kernel(
        _wdeg_body,
        out_type=[
            jax.ShapeDtypeStruct((_E,), _f32),
            jax.ShapeDtypeStruct((2 * _N,), _f32),
        ],
        mesh=_mesh(),
        scratch_types=[
            pltpu.VMEM((_EW_CH,), _i32),
            pltpu.VMEM((_EW_CH,), _i32),
            pltpu.VMEM((_EW_CH,), _i32),
            pltpu.VMEM((_EW_CH,), _f32),
            pltpu.VMEM((_EW_TAIL,), _i32),
            pltpu.VMEM((_EW_TAIL,), _i32),
            pltpu.VMEM((_EW_TAIL,), _i32),
            pltpu.VMEM((_EW_TAIL,), _f32),
            pltpu.VMEM((3072,), _f32),
            pltpu.VMEM_SHARED((_N,), _f32),
        ],
    )
    return k(row, col, tflat)


_AG_TILE = _E // _NT
_AG_CH = 64
_AG_NCH = _AG_TILE // _AG_CH
_ROWS_T = _N // _NT


def _agg_body(row_h, col_h, w_h, dinv_h, hf_h, agg_h,
              rbuf, cbuf, gbuf, wbuf, nbuf, stage, zbuf, dinvb, acc):
    ci = lax.axis_index("c")
    s = lax.axis_index("s")
    coff = ci * _N
    tbase = s * _AG_TILE

    pltpu.sync_copy(dinv_h, dinvb)

    @pl.loop(0, 128)
    def _z(i):
        for q in range(4):
            zbuf[i, pl.ds(q * 16, 16)] = jnp.zeros((16,), _f32)

    rbase = s * _ROWS_T
    for kk in range(11):
        pltpu.sync_copy(zbuf, acc.at[pl.ds(rbase + kk * 128, 128)])
    pltpu.sync_copy(zbuf.at[pl.ds(0, 108)],
                    acc.at[pl.ds(rbase + 1408, 108)])

    plsc.subcore_barrier()

    @pl.loop(0, _AG_NCH)
    def _c(i):
        base = tbase + i * _AG_CH
        pltpu.sync_copy(row_h.at[pl.ds(base, _AG_CH)], rbuf)
        pltpu.sync_copy(col_h.at[pl.ds(base, _AG_CH)], cbuf)
        pltpu.sync_copy(w_h.at[pl.ds(base, _AG_CH)], wbuf)
        for g in range(_AG_CH // 16):
            sl = pl.ds(g * 16, 16)
            rv = rbuf[sl]
            cv = cbuf[sl]
            dr = plsc.load_gather(dinvb, [rv])
            dc = plsc.load_gather(dinvb, [cv])
            nbuf[sl] = dr * wbuf[sl] * dc
            gbuf[sl] = rv + coff
        pltpu.sync_copy(hf_h.at[gbuf], stage)
        for e in range(_AG_CH):
            esplat = lax.full((16,), e, _i32)
            nb = plsc.load_gather(nbuf, [esplat])
            for q in range(4):
                sl2 = pl.ds(q * 16, 16)
                stage[e, sl2] = stage[e, sl2] * nb
        pltpu.sync_copy(stage, acc.at[cbuf], add=True)

    plsc.subcore_barrier()

    pltpu.sync_copy(acc.at[pl.ds(rbase, _ROWS_T)],
                    agg_h.at[pl.ds(coff + rbase, _ROWS_T)])


def _sc_agg(row, col, w, dinv, hflat):
    k = pl.kernel(
        _agg_body,
        out_type=jax.ShapeDtypeStruct((2 * _N, _HF), _f32),
        mesh=_mesh(),
        scratch_types=[
            pltpu.VMEM((_AG_CH,), _i32),
            pltpu.VMEM((_AG_CH,), _i32),
            pltpu.VMEM((_AG_CH,), _i32),
            pltpu.VMEM((_AG_CH,), _f32),
            pltpu.VMEM((_AG_CH,), _f32),
            pltpu.VMEM((_AG_CH, _HF), _f32),
            pltpu.VMEM((128, _HF), _f32),
            pltpu.VMEM((_N,), _f32),
            pltpu.VMEM_SHARED((_N, _HF), _f32),
        ],
    )
    return k(row, col, w, dinv, hflat)


_PL_FULL = _N // 128
_PL_TAIL = _N - _PL_FULL * 128


def _pool_body(xf_h, batch_h, sums_h, cnt_h,
               bbuf, btail, stage, onesv, zbv, zcv, sumS, cntS):
    ci = lax.axis_index("c")
    s = lax.axis_index("s")
    coff = ci * _N

    @pl.loop(0, 64)
    def _z(i):
        for q in range(4):
            zbv[i, pl.ds(q * 16, 16)] = jnp.zeros((16,), _f32)

    @pl.loop(0, 128)
    def _o(i):
        onesv[i, pl.ds(0, 16)] = jnp.ones((16,), _f32)

    @pl.loop(0, 64)
    def _zc(i):
        zcv[i, pl.ds(0, 16)] = jnp.zeros((16,), _f32)

    @pl.when(s == 0)
    def _zero():
        pltpu.sync_copy(zbv, sumS)

    @pl.when(jnp.logical_and(s == 0, ci == 0))
    def _zeroc():
        pltpu.sync_copy(zcv, cntS)

    plsc.subcore_barrier()

    nch = (204 - s) // 16

    @pl.loop(0, nch)
    def _c(k):
        j = s + k * _NT
        base = j * 128
        pltpu.sync_copy(batch_h.at[pl.ds(base, 128)], bbuf)
        pltpu.sync_copy(xf_h.at[pl.ds(coff + base, 128)], stage)
        pltpu.sync_copy(stage, sumS.at[bbuf], add=True)

        @pl.when(ci == 0)
        def _cnt():
            pltpu.sync_copy(onesv, cntS.at[bbuf], add=True)

    @pl.when(s == _NT - 1)
    def _tail():
        base = _PL_FULL * 128
        pltpu.sync_copy(batch_h.at[pl.ds(base, _PL_TAIL)], btail)
        pltpu.sync_copy(xf_h.at[pl.ds(coff + base, _PL_TAIL)],
                        stage.at[pl.ds(0, _PL_TAIL)])
        pltpu.sync_copy(stage.at[pl.ds(0, _PL_TAIL)],
                        sumS.at[btail], add=True)

        @pl.when(ci == 0)
        def _cntt():
            pltpu.sync_copy(onesv.at[pl.ds(0, _PL_TAIL)],
                            cntS.at[btail], add=True)

    plsc.subcore_barrier()

    @pl.when(s == 0)
    def _out():
        pltpu.sync_copy(sumS, sums_h.at[ci])

    @pl.when(jnp.logical_and(s == 0, ci == 0))
    def _outc():
        pltpu.sync_copy(cntS, cnt_h)


def _sc_pool(xflat, batch):
    k = pl.kernel(
        _pool_body,
        out_type=[
            jax.ShapeDtypeStruct((2, _B, _HF), _f32),
            jax.ShapeDtypeStruct((_B, 16), _f32),
        ],
        mesh=_mesh(),
        scratch_types=[
            pltpu.VMEM((128,), _i32),
            pltpu.VMEM((_PL_TAIL,), _i32),
            pltpu.VMEM((128, _HF), _f32),
            pltpu.VMEM((128, 16), _f32),
            pltpu.VMEM((_B, _HF), _f32),
            pltpu.VMEM((_B, 16), _f32),
            pltpu.VMEM_SHARED((_B, _HF), _f32),
            pltpu.VMEM_SHARED((_B, 16), _f32),
        ],
    )
    return k(xflat, batch)


def kernel(x, edge_index, edge_weight, batch, W1, b1, W2, b2, We, be, Wf, bf, lew):
    del edge_weight
    row = edge_index[0]
    col = edge_index[1]

    lew_pad = jnp.pad(lew, ((0, _RP - _R), (0, _RP - _R)))
    tflat = _tc_table(lew_pad).reshape(-1)

    w, degpf = _sc_wdeg(row, col, tflat)
    dinv8, dinv28 = _tc_deg(degpf.reshape(16, _BN))
    dinv = dinv8.reshape(-1)
    d2col = dinv28.reshape(-1, 1)

    h1, xe = _tc_mm(x, W1, We, be.reshape(1, _D))
    agg1 = _sc_agg(row, col, w, dinv, h1.reshape(2 * _N, _HF))
    x1, h2 = _tc_l1(agg1.reshape(2, _N, _HF), h1, xe, d2col,
                    b1.reshape(1, _D), W2)
    agg2 = _sc_agg(row, col, w, dinv, h2.reshape(2 * _N, _HF))
    x2 = _tc_l2(agg2.reshape(2, _N, _HF), h2, x1, d2col, b2.reshape(1, _D))

    sums, cnt16 = _sc_pool(x2.reshape(2 * _N, _HF), batch)
    return _tc_pred(sums, cnt16, Wf, bf.reshape(1, 1))


# speedup vs baseline: 6.1092x; 6.1092x over previous
import functools

import jax
import jax.numpy as jnp
from jax import lax
from jax.experimental import pallas as pl
from jax.experimental.pallas import tpu as pltpu
from jax.experimental.pallas import tpu_sc as plsc

_N = 24256
_E = 388096
_D = 128
_B = 64
_R = 379
_RP = 384
_HF = 64
_NT = 16
_NC = 2
_BN = 3032

_f32 = jnp.float32
_i32 = jnp.int32


def _mesh():
    return plsc.VectorSubcoreMesh(core_axis_name="c", subcore_axis_name="s")


def _table_body(lew_ref, t_ref):
    a = lew_ref[...]
    t_ref[...] = 2.0 * jax.nn.sigmoid((a + a.T) * 0.5)


def _tc_table(lew_pad):
    return pl.pallas_call(
        _table_body,
        out_shape=jax.ShapeDtypeStruct((_RP, _RP), _f32),
    )(lew_pad)


def _deg_body(dp_ref, dinv_ref):
    dp = dp_ref[...]
    deg = dp[:8] + dp[8:] + 1.0
    dinv_ref[...] = lax.rsqrt(deg)


def _tc_deg(degp):
    return pl.pallas_call(
        _deg_body,
        out_shape=jax.ShapeDtypeStruct((8, _BN), _f32),
    )(degp)


def _mm_body(x_ref, w1_ref, we_ref, be_ref, dv_ref, hs1_ref, xe_ref):
    xb = x_ref[...]
    dv = dv_ref[...]
    hs1 = jnp.dot(xb, w1_ref[...], preferred_element_type=_f32) * dv
    xe = jnp.dot(xb, we_ref[...], preferred_element_type=_f32) + be_ref[...]
    xe = jnp.maximum(xe, 0.0)
    hs1_ref[0] = hs1[:, :_HF]
    hs1_ref[1] = hs1[:, _HF:]
    xe_ref[0] = xe[:, :_HF]
    xe_ref[1] = xe[:, _HF:]


def _tc_mm(x, w1, we, be_row, dvcol):
    grid = _N // _BN
    return pl.pallas_call(
        _mm_body,
        grid=(grid,),
        in_specs=[
            pl.BlockSpec((_BN, _D), lambda i: (i, 0)),
            pl.BlockSpec((_D, _D), lambda i: (0, 0)),
            pl.BlockSpec((_D, _D), lambda i: (0, 0)),
            pl.BlockSpec((1, _D), lambda i: (0, 0)),
            pl.BlockSpec((_BN, 1), lambda i: (i, 0)),
        ],
        out_specs=[
            pl.BlockSpec((2, _BN, _HF), lambda i: (0, i, 0)),
            pl.BlockSpec((2, _BN, _HF), lambda i: (0, i, 0)),
        ],
        out_shape=[
            jax.ShapeDtypeStruct((2, _N, _HF), _f32),
            jax.ShapeDtypeStruct((2, _N, _HF), _f32),
        ],
    )(x, w1, we, be_row, dvcol)


def _l1_body(agg_ref, hs1_ref, xe_ref, dv_ref, b1_ref, w2_ref, x1_ref, hs2_ref):
    dv = dv_ref[...]
    b1 = b1_ref[...]
    x1h = []
    for t in range(2):
        o = (agg_ref[t] + hs1_ref[t]) * dv + b1[:, _HF * t:_HF * (t + 1)]
        x1h.append(jnp.maximum(o, 0.0) + xe_ref[t])
    w2 = w2_ref[...]
    hs2 = (jnp.dot(x1h[0], w2[:_HF, :], preferred_element_type=_f32)
           + jnp.dot(x1h[1], w2[_HF:, :], preferred_element_type=_f32)) * dv
    x1_ref[0] = x1h[0]
    x1_ref[1] = x1h[1]
    hs2_ref[0] = hs2[:, :_HF]
    hs2_ref[1] = hs2[:, _HF:]


def _tc_l1(agg1, h1, xe, d2col, b1_row, w2):
    grid = _N // _BN
    half_spec = pl.BlockSpec((2, _BN, _HF), lambda i: (0, i, 0))
    return pl.pallas_call(
        _l1_body,
        grid=(grid,),
        in_specs=[
            half_spec,
            half_spec,
            half_spec,
            pl.BlockSpec((_BN, 1), lambda i: (i, 0)),
            pl.BlockSpec((1, _D), lambda i: (0, 0)),
            pl.BlockSpec((_D, _D), lambda i: (0, 0)),
        ],
        out_specs=[half_spec, half_spec],
        out_shape=[
            jax.ShapeDtypeStruct((2, _N, _HF), _f32),
            jax.ShapeDtypeStruct((2, _N, _HF), _f32),
        ],
    )(agg1, h1, xe, d2col, b1_row, w2)


def _l2_body(agg_ref, hs2_ref, x1_ref, dv_ref, b2_ref, x2_ref):
    dv = dv_ref[...]
    b2 = b2_ref[...]
    for t in range(2):
        o = (agg_ref[t] + hs2_ref[t]) * dv + b2[:, _HF * t:_HF * (t + 1)]
        x2_ref[t] = jnp.maximum(o, 0.0) + x1_ref[t]


def _tc_l2(agg2, h2, x1, d2col, b2_row):
    grid = _N // _BN
    half_spec = pl.BlockSpec((2, _BN, _HF), lambda i: (0, i, 0))
    return pl.pallas_call(
        _l2_body,
        grid=(grid,),
        in_specs=[
            half_spec,
            half_spec,
            half_spec,
            pl.BlockSpec((_BN, 1), lambda i: (i, 0)),
            pl.BlockSpec((1, _D), lambda i: (0, 0)),
        ],
        out_specs=half_spec,
        out_shape=jax.ShapeDtypeStruct((2, _N, _HF), _f32),
    )(agg2, h2, x1, d2col, b2_row)


def _pred_body(sums_ref, cnt_ref, wf_ref, bf_ref, out_ref):
    cnt = jnp.maximum(cnt_ref[...][:, 0:1], 1.0)
    wf = wf_ref[...]
    p0 = sums_ref[0] / cnt
    p1 = sums_ref[1] / cnt
    out_ref[...] = (jnp.dot(p0, wf[:_HF, :], preferred_element_type=_f32)
                    + jnp.dot(p1, wf[_HF:, :], preferred_element_type=_f32)
                    + bf_ref[...])


def _tc_pred(sums, cnt16, wf, bf_row):
    return pl.pallas_call(
        _pred_body,
        out_shape=jax.ShapeDtypeStruct((_B, 1), _f32),
    )(sums, cnt16, wf, bf_row)


_EW_TILE = _E // (_NC * _NT)
_EW_CH = 128
_EW_FULL = _EW_TILE // _EW_CH
_EW_TAIL = _EW_TILE - _EW_FULL * _EW_CH

_DEG_SL = _N // 8


def _wdeg_body(row_h, col_h, tflat_h, w_h, degp_h,
               rbuf, cbuf, fbuf, wstage, rtb, ctb, ftb, wtb, zb, degS):
    ci = lax.axis_index("c")
    s = lax.axis_index("s")
    wid = ci * _NT + s
    tbase = wid * _EW_TILE

    @pl.loop(0, 192)
    def _z(i):
        zb[pl.ds(i * 16, 16)] = jnp.zeros((16,), _f32)

    @pl.when(s < 8)
    def _zdeg():
        pltpu.sync_copy(zb.at[pl.ds(0, _DEG_SL)],
                        degS.at[pl.ds(s * _DEG_SL, _DEG_SL)])

    plsc.subcore_barrier()

    def _chunk(base, ch, rb, cb, fb, wst):
        pltpu.sync_copy(row_h.at[pl.ds(base, ch)], rb)
        pltpu.sync_copy(col_h.at[pl.ds(base, ch)], cb)
        for g in range(ch // 16):
            sl = pl.ds(g * 16, 16)
            rv = rb[sl]
            cv = cb[sl]
            fb[sl] = (rv % _R) * _RP + (cv % _R)
        pltpu.sync_copy(tflat_h.at[fb], wst)
        pltpu.sync_copy(wst, w_h.at[pl.ds(base, ch)])
        pltpu.sync_copy(wst, degS.at[cb], add=True)

    @pl.loop(0, _EW_FULL)
    def _c(i):
        _chunk(tbase + i * _EW_CH, _EW_CH, rbuf, cbuf, fbuf, wstage)

    _chunk(tbase + _EW_FULL * _EW_CH, _EW_TAIL, rtb, ctb, ftb, wtb)

    plsc.subcore_barrier()

    @pl.when(s < 8)
    def _out():
        pltpu.sync_copy(degS.at[pl.ds(s * _DEG_SL, _DEG_SL)],
                        zb.at[pl.ds(0, _DEG_SL)])
        pltpu.sync_copy(zb.at[pl.ds(0, _DEG_SL)],
                        degp_h.at[pl.ds(ci * _N + s * _DEG_SL, _DEG_SL)])


def _sc_wdeg(row, col, tflat):
    k = pl.kernel(
        _wdeg_body,
        out_type=[
            jax.ShapeDtypeStruct((_E,), _f32),
            jax.ShapeDtypeStruct((2 * _N,), _f32),
        ],
        mesh=_mesh(),
        compiler_params=pltpu.CompilerParams(use_tc_tiling_on_sc=False),
        scratch_types=[
            pltpu.VMEM((_EW_CH,), _i32),
            pltpu.VMEM((_EW_CH,), _i32),
            pltpu.VMEM((_EW_CH,), _i32),
            pltpu.VMEM((_EW_CH,), _f32),
            pltpu.VMEM((_EW_TAIL,), _i32),
            pltpu.VMEM((_EW_TAIL,), _i32),
            pltpu.VMEM((_EW_TAIL,), _i32),
            pltpu.VMEM((_EW_TAIL,), _f32),
            pltpu.VMEM((3072,), _f32),
            pltpu.VMEM_SHARED((_N,), _f32),
        ],
    )
    return k(row, col, tflat)


_AG_TILE = _E // _NT
_AG_CH = 64
_AG_NCH = _AG_TILE // _AG_CH
_ROWS_T = 1520
_ROWS_LAST = _N - 15 * _ROWS_T


def _agg_body(row_h, col_h, w_h, hf_h, agg_h,
              rbuf, cbuf, gbuf, wbuf, stage, zbuf, acc):
    ci = lax.axis_index("c")
    s = lax.axis_index("s")
    coff = ci * _N
    tbase = s * _AG_TILE

    @pl.loop(0, 128)
    def _z(i):
        for q in range(4):
            zbuf[i, pl.ds(q * 16, 16)] = jnp.zeros((16,), _f32)

    rbase = pl.multiple_of(s * _ROWS_T, 8)
    for kk in range(11):
        pltpu.sync_copy(zbuf, acc.at[pl.ds(rbase + kk * 128, 128)])

    @pl.when(s < _NT - 1)
    def _ztail():
        pltpu.sync_copy(zbuf.at[pl.ds(0, 112)],
                        acc.at[pl.ds(rbase + 1408, 112)])

    @pl.when(s == _NT - 1)
    def _ztail_last():
        pltpu.sync_copy(zbuf.at[pl.ds(0, 48)],
                        acc.at[pl.ds(rbase + 1408, 48)])

    plsc.subcore_barrier()

    @pl.loop(0, _AG_NCH)
    def _c(i):
        base = tbase + i * _AG_CH
        pltpu.sync_copy(row_h.at[pl.ds(base, _AG_CH)], rbuf)
        pltpu.sync_copy(col_h.at[pl.ds(base, _AG_CH)], cbuf)
        pltpu.sync_copy(w_h.at[pl.ds(base, _AG_CH)], wbuf)
        for g in range(_AG_CH // 16):
            sl = pl.ds(g * 16, 16)
            gbuf[sl] = rbuf[sl] + coff
        pltpu.sync_copy(hf_h.at[gbuf], stage)
        for g4 in range(_AG_CH // 16):
            wv = wbuf[pl.ds(g4 * 16, 16)]
            for l in range(16):
                e = g4 * 16 + l
                nb = wv.at[lax.full((16,), l, _i32)].get(
                    mode="promise_in_bounds")
                for q in range(4):
                    sl2 = pl.ds(q * 16, 16)
                    stage[e, sl2] = stage[e, sl2] * nb
        pltpu.sync_copy(stage, acc.at[cbuf], add=True)

    plsc.subcore_barrier()

    obase = pl.multiple_of(coff + rbase, 8)
    for kk in range(11):
        pltpu.sync_copy(acc.at[pl.ds(rbase + kk * 128, 128)], zbuf)
        pltpu.sync_copy(zbuf, agg_h.at[pl.ds(obase + kk * 128, 128)])

    @pl.when(s < _NT - 1)
    def _otail():
        pltpu.sync_copy(acc.at[pl.ds(rbase + 1408, 112)],
                        zbuf.at[pl.ds(0, 112)])
        pltpu.sync_copy(zbuf.at[pl.ds(0, 112)],
                        agg_h.at[pl.ds(obase + 1408, 112)])

    @pl.when(s == _NT - 1)
    def _otail_last():
        pltpu.sync_copy(acc.at[pl.ds(rbase + 1408, 48)],
                        zbuf.at[pl.ds(0, 48)])
        pltpu.sync_copy(zbuf.at[pl.ds(0, 48)],
                        agg_h.at[pl.ds(obase + 1408, 48)])


def _sc_agg(row, col, w, hflat):
    k = pl.kernel(
        _agg_body,
        out_type=jax.ShapeDtypeStruct((2 * _N, _HF), _f32),
        mesh=_mesh(),
        compiler_params=pltpu.CompilerParams(use_tc_tiling_on_sc=False),
        scratch_types=[
            pltpu.VMEM((_AG_CH,), _i32),
            pltpu.VMEM((_AG_CH,), _i32),
            pltpu.VMEM((_AG_CH,), _i32),
            pltpu.VMEM((_AG_CH,), _f32),
            pltpu.VMEM((_AG_CH, _HF), _f32),
            pltpu.VMEM((128, _HF), _f32),
            pltpu.VMEM_SHARED((_N, _HF), _f32),
        ],
    )
    return k(row, col, w, hflat)


_PL_FULL = _N // 128
_PL_TAIL = _N - _PL_FULL * 128


def _pool_body(xf_h, batch_h, sums_h, cnt_h,
               bbuf, btail, stage, onesv, zbv, zcv, sumS, cntS):
    ci = lax.axis_index("c")
    s = lax.axis_index("s")
    coff = ci * _N

    @pl.loop(0, 64)
    def _z(i):
        for q in range(4):
            zbv[i, pl.ds(q * 16, 16)] = jnp.zeros((16,), _f32)

    @pl.loop(0, 128)
    def _o(i):
        onesv[i, pl.ds(0, 16)] = jnp.ones((16,), _f32)

    @pl.loop(0, 64)
    def _zc(i):
        zcv[i, pl.ds(0, 16)] = jnp.zeros((16,), _f32)

    @pl.when(s == 0)
    def _zero():
        pltpu.sync_copy(zbv, sumS)

    @pl.when(jnp.logical_and(s == 0, ci == 0))
    def _zeroc():
        pltpu.sync_copy(zcv, cntS)

    plsc.subcore_barrier()

    nch = (204 - s) // 16

    @pl.loop(0, nch)
    def _c(k):
        j = s + k * _NT
        base = j * 128
        pltpu.sync_copy(batch_h.at[pl.ds(base, 128)], bbuf)
        pltpu.sync_copy(xf_h.at[pl.ds(coff + base, 128)], stage)
        pltpu.sync_copy(stage, sumS.at[bbuf], add=True)

        @pl.when(ci == 0)
        def _cnt():
            pltpu.sync_copy(onesv, cntS.at[bbuf], add=True)

    @pl.when(s == _NT - 1)
    def _tail():
        base = _PL_FULL * 128
        pltpu.sync_copy(batch_h.at[pl.ds(base, _PL_TAIL)], btail)
        pltpu.sync_copy(xf_h.at[pl.ds(coff + base, _PL_TAIL)],
                        stage.at[pl.ds(0, _PL_TAIL)])
        pltpu.sync_copy(stage.at[pl.ds(0, _PL_TAIL)],
                        sumS.at[btail], add=True)

        @pl.when(ci == 0)
        def _cntt():
            pltpu.sync_copy(onesv.at[pl.ds(0, _PL_TAIL)],
                            cntS.at[btail], add=True)

    plsc.subcore_barrier()

    @pl.when(s == 0)
    def _out():
        pltpu.sync_copy(sumS, zbv)
        pltpu.sync_copy(zbv, sums_h.at[ci])

    @pl.when(jnp.logical_and(s == 0, ci == 0))
    def _outc():
        pltpu.sync_copy(cntS, zcv)
        pltpu.sync_copy(zcv, cnt_h)


def _sc_pool(xflat, batch):
    k = pl.kernel(
        _pool_body,
        out_type=[
            jax.ShapeDtypeStruct((2, _B, _HF), _f32),
            jax.ShapeDtypeStruct((_B, 16), _f32),
        ],
        mesh=_mesh(),
        compiler_params=pltpu.CompilerParams(use_tc_tiling_on_sc=False),
        scratch_types=[
            pltpu.VMEM((128,), _i32),
            pltpu.VMEM((_PL_TAIL,), _i32),
            pltpu.VMEM((128, _HF), _f32),
            pltpu.VMEM((128, 16), _f32),
            pltpu.VMEM((_B, _HF), _f32),
            pltpu.VMEM((_B, 16), _f32),
            pltpu.VMEM_SHARED((_B, _HF), _f32),
            pltpu.VMEM_SHARED((_B, 16), _f32),
        ],
    )
    return k(xflat, batch)


def kernel(x, edge_index, edge_weight, batch, W1, b1, W2, b2, We, be, Wf, bf, lew):
    del edge_weight
    row = edge_index[0]
    col = edge_index[1]

    lew_pad = jnp.pad(lew, ((0, _RP - _R), (0, _RP - _R)))
    tflat = _tc_table(lew_pad).reshape(-1)

    w, degpf = _sc_wdeg(row, col, tflat)
    dinv8 = _tc_deg(degpf.reshape(16, _BN))
    dvcol = dinv8.reshape(-1, 1)

    hs1, xe = _tc_mm(x, W1, We, be.reshape(1, _D), dvcol)
    agg1 = _sc_agg(row, col, w, hs1.reshape(2 * _N, _HF))
    x1, hs2 = _tc_l1(agg1.reshape(2, _N, _HF), hs1, xe, dvcol,
                     b1.reshape(1, _D), W2)
    agg2 = _sc_agg(row, col, w, hs2.reshape(2 * _N, _HF))
    x2 = _tc_l2(agg2.reshape(2, _N, _HF), hs2, x1, dvcol, b2.reshape(1, _D))

    sums, cnt16 = _sc_pool(x2.reshape(2 * _N, _HF), batch)
    return _tc_pred(sums, cnt16, Wf, bf.reshape(1, 1))


# speedup vs baseline: 11.9627x; 1.9582x over previous
import functools

import jax
import jax.numpy as jnp
from jax import lax
from jax.experimental import pallas as pl
from jax.experimental.pallas import tpu as pltpu
from jax.experimental.pallas import tpu_sc as plsc

_N = 24256
_E = 388096
_D = 128
_B = 64
_R = 379
_RP = 384
_HF = 64
_NT = 16
_NC = 2
_BN = 3032

_f32 = jnp.float32
_i32 = jnp.int32


def _mesh():
    return plsc.VectorSubcoreMesh(core_axis_name="c", subcore_axis_name="s")


def _table_body(lew_ref, t_ref):
    a = lew_ref[...]
    t_ref[...] = 2.0 * jax.nn.sigmoid((a + a.T) * 0.5)


def _tc_table(lew_pad):
    return pl.pallas_call(
        _table_body,
        out_shape=jax.ShapeDtypeStruct((_RP, _RP), _f32),
    )(lew_pad)


def _deg_body(dp_ref, dinv_ref):
    dp = dp_ref[...]
    deg = dp[:8] + dp[8:] + 1.0
    dinv_ref[...] = lax.rsqrt(deg)


def _tc_deg(degp):
    return pl.pallas_call(
        _deg_body,
        out_shape=jax.ShapeDtypeStruct((8, _BN), _f32),
    )(degp)


def _mm_body(x_ref, w1_ref, we_ref, be_ref, dv_ref, hs1_ref, xe_ref):
    xb = x_ref[...]
    dv = dv_ref[...]
    hs1 = jnp.dot(xb, w1_ref[...], preferred_element_type=_f32) * dv
    xe = jnp.dot(xb, we_ref[...], preferred_element_type=_f32) + be_ref[...]
    xe = jnp.maximum(xe, 0.0)
    hs1_ref[0] = hs1[:, :_HF]
    hs1_ref[1] = hs1[:, _HF:]
    xe_ref[0] = xe[:, :_HF]
    xe_ref[1] = xe[:, _HF:]


def _tc_mm(x, w1, we, be_row, dvcol):
    grid = _N // _BN
    return pl.pallas_call(
        _mm_body,
        grid=(grid,),
        in_specs=[
            pl.BlockSpec((_BN, _D), lambda i: (i, 0)),
            pl.BlockSpec((_D, _D), lambda i: (0, 0)),
            pl.BlockSpec((_D, _D), lambda i: (0, 0)),
            pl.BlockSpec((1, _D), lambda i: (0, 0)),
            pl.BlockSpec((_BN, 1), lambda i: (i, 0)),
        ],
        out_specs=[
            pl.BlockSpec((2, _BN, _HF), lambda i: (0, i, 0)),
            pl.BlockSpec((2, _BN, _HF), lambda i: (0, i, 0)),
        ],
        out_shape=[
            jax.ShapeDtypeStruct((2, _N, _HF), _f32),
            jax.ShapeDtypeStruct((2, _N, _HF), _f32),
        ],
    )(x, w1, we, be_row, dvcol)


def _l1_body(agg_ref, hs1_ref, xe_ref, dv_ref, b1_ref, w2_ref, x1_ref, hs2_ref):
    dv = dv_ref[...]
    b1 = b1_ref[...]
    x1h = []
    for t in range(2):
        o = (agg_ref[t] + hs1_ref[t]) * dv + b1[:, _HF * t:_HF * (t + 1)]
        x1h.append(jnp.maximum(o, 0.0) + xe_ref[t])
    w2 = w2_ref[...]
    hs2 = (jnp.dot(x1h[0], w2[:_HF, :], preferred_element_type=_f32)
           + jnp.dot(x1h[1], w2[_HF:, :], preferred_element_type=_f32)) * dv
    x1_ref[0] = x1h[0]
    x1_ref[1] = x1h[1]
    hs2_ref[0] = hs2[:, :_HF]
    hs2_ref[1] = hs2[:, _HF:]


def _tc_l1(agg1, h1, xe, d2col, b1_row, w2):
    grid = _N // _BN
    half_spec = pl.BlockSpec((2, _BN, _HF), lambda i: (0, i, 0))
    return pl.pallas_call(
        _l1_body,
        grid=(grid,),
        in_specs=[
            half_spec,
            half_spec,
            half_spec,
            pl.BlockSpec((_BN, 1), lambda i: (i, 0)),
            pl.BlockSpec((1, _D), lambda i: (0, 0)),
            pl.BlockSpec((_D, _D), lambda i: (0, 0)),
        ],
        out_specs=[half_spec, half_spec],
        out_shape=[
            jax.ShapeDtypeStruct((2, _N, _HF), _f32),
            jax.ShapeDtypeStruct((2, _N, _HF), _f32),
        ],
    )(agg1, h1, xe, d2col, b1_row, w2)


def _l2_body(agg_ref, hs2_ref, x1_ref, dv_ref, b2_ref, x2_ref):
    dv = dv_ref[...]
    b2 = b2_ref[...]
    for t in range(2):
        o = (agg_ref[t] + hs2_ref[t]) * dv + b2[:, _HF * t:_HF * (t + 1)]
        x2_ref[t] = jnp.maximum(o, 0.0) + x1_ref[t]


def _tc_l2(agg2, h2, x1, d2col, b2_row):
    grid = _N // _BN
    half_spec = pl.BlockSpec((2, _BN, _HF), lambda i: (0, i, 0))
    return pl.pallas_call(
        _l2_body,
        grid=(grid,),
        in_specs=[
            half_spec,
            half_spec,
            half_spec,
            pl.BlockSpec((_BN, 1), lambda i: (i, 0)),
            pl.BlockSpec((1, _D), lambda i: (0, 0)),
        ],
        out_specs=half_spec,
        out_shape=jax.ShapeDtypeStruct((2, _N, _HF), _f32),
    )(agg2, h2, x1, d2col, b2_row)


def _pred_body(sums_ref, cnt_ref, wf_ref, bf_ref, out_ref):
    cnt = jnp.maximum(cnt_ref[...][:, 0:1], 1.0)
    wf = wf_ref[...]
    p0 = sums_ref[0] / cnt
    p1 = sums_ref[1] / cnt
    out_ref[...] = (jnp.dot(p0, wf[:_HF, :], preferred_element_type=_f32)
                    + jnp.dot(p1, wf[_HF:, :], preferred_element_type=_f32)
                    + bf_ref[...])


def _tc_pred(sums, cnt16, wf, bf_row):
    return pl.pallas_call(
        _pred_body,
        out_shape=jax.ShapeDtypeStruct((_B, 1), _f32),
    )(sums, cnt16, wf, bf_row)


_EW_TILE = _E // (_NC * _NT)
_EW_CH = 128
_EW_FULL = _EW_TILE // _EW_CH
_EW_TAIL = _EW_TILE - _EW_FULL * _EW_CH

_DEG_SL = _N // 8


def _wdeg_body(row_h, col_h, tflat_h, w_h, degp_h,
               rbuf, cbuf, fbuf, wstage, rtb, ctb, ftb, wtb, zb, degS):
    ci = lax.axis_index("c")
    s = lax.axis_index("s")
    wid = ci * _NT + s
    tbase = wid * _EW_TILE

    @pl.loop(0, 192)
    def _z(i):
        zb[pl.ds(i * 16, 16)] = jnp.zeros((16,), _f32)

    @pl.when(s < 8)
    def _zdeg():
        pltpu.sync_copy(zb.at[pl.ds(0, _DEG_SL)],
                        degS.at[pl.ds(s * _DEG_SL, _DEG_SL)])

    plsc.subcore_barrier()

    def _chunk(base, ch, rb, cb, fb, wst):
        pltpu.sync_copy(row_h.at[pl.ds(base, ch)], rb)
        pltpu.sync_copy(col_h.at[pl.ds(base, ch)], cb)
        for g in range(ch // 16):
            sl = pl.ds(g * 16, 16)
            rv = rb[sl]
            cv = cb[sl]
            fb[sl] = (rv % _R) * _RP + (cv % _R)
        pltpu.sync_copy(tflat_h.at[fb], wst)
        pltpu.sync_copy(wst, w_h.at[pl.ds(base, ch)])
        pltpu.sync_copy(wst, degS.at[cb], add=True)

    @pl.loop(0, _EW_FULL)
    def _c(i):
        _chunk(tbase + i * _EW_CH, _EW_CH, rbuf, cbuf, fbuf, wstage)

    _chunk(tbase + _EW_FULL * _EW_CH, _EW_TAIL, rtb, ctb, ftb, wtb)

    plsc.subcore_barrier()

    @pl.when(s < 8)
    def _out():
        pltpu.sync_copy(degS.at[pl.ds(s * _DEG_SL, _DEG_SL)],
                        zb.at[pl.ds(0, _DEG_SL)])
        pltpu.sync_copy(zb.at[pl.ds(0, _DEG_SL)],
                        degp_h.at[pl.ds(ci * _N + s * _DEG_SL, _DEG_SL)])


def _sc_wdeg(row, col, tflat):
    k = pl.kernel(
        _wdeg_body,
        out_type=[
            jax.ShapeDtypeStruct((_E,), _f32),
            jax.ShapeDtypeStruct((2 * _N,), _f32),
        ],
        mesh=_mesh(),
        compiler_params=pltpu.CompilerParams(use_tc_tiling_on_sc=False),
        scratch_types=[
            pltpu.VMEM((_EW_CH,), _i32),
            pltpu.VMEM((_EW_CH,), _i32),
            pltpu.VMEM((_EW_CH,), _i32),
            pltpu.VMEM((_EW_CH,), _f32),
            pltpu.VMEM((_EW_TAIL,), _i32),
            pltpu.VMEM((_EW_TAIL,), _i32),
            pltpu.VMEM((_EW_TAIL,), _i32),
            pltpu.VMEM((_EW_TAIL,), _f32),
            pltpu.VMEM((3072,), _f32),
            pltpu.VMEM_SHARED((_N,), _f32),
        ],
    )
    return k(row, col, tflat)


_AG_TILE = _E // _NT
_AG_CH = 64
_AG_NCH = _AG_TILE // _AG_CH
_ROWS_T = 1520
_ROWS_LAST = _N - 15 * _ROWS_T


def _agg_body(row_h, col_h, w_h, hf_h, agg_h,
              rbuf, cbuf, gbuf, wbuf, stage,
              rbuf2, cbuf2, gbuf2, wbuf2, stage2,
              semi0, semi1, semg0, semg1, zbuf, acc):
    ci = lax.axis_index("c")
    s = lax.axis_index("s")
    coff = ci * _N
    tbase = s * _AG_TILE

    @pl.loop(0, 128)
    def _z(i):
        for q in range(4):
            zbuf[i, pl.ds(q * 16, 16)] = jnp.zeros((16,), _f32)

    rbase = pl.multiple_of(s * _ROWS_T, 8)
    for kk in range(11):
        pltpu.sync_copy(zbuf, acc.at[pl.ds(rbase + kk * 128, 128)])

    @pl.when(s < _NT - 1)
    def _ztail():
        pltpu.sync_copy(zbuf.at[pl.ds(0, 112)],
                        acc.at[pl.ds(rbase + 1408, 112)])

    @pl.when(s == _NT - 1)
    def _ztail_last():
        pltpu.sync_copy(zbuf.at[pl.ds(0, 48)],
                        acc.at[pl.ds(rbase + 1408, 48)])

    plsc.subcore_barrier()

    rb = (rbuf, rbuf2)
    cb = (cbuf, cbuf2)
    gb = (gbuf, gbuf2)
    wb = (wbuf, wbuf2)
    st = (stage, stage2)
    semi = (semi0, semi1)
    semg = (semg0, semg1)

    def start_idx(i, bs):
        base = tbase + i * _AG_CH
        pltpu.async_copy(row_h.at[pl.ds(base, _AG_CH)], rb[bs], semi[bs])
        pltpu.async_copy(col_h.at[pl.ds(base, _AG_CH)], cb[bs], semi[bs])
        pltpu.async_copy(w_h.at[pl.ds(base, _AG_CH)], wb[bs], semi[bs])

    def wait_idx(bs):
        pltpu.make_async_copy(row_h.at[pl.ds(tbase, _AG_CH)], rb[bs],
                              semi[bs]).wait()
        pltpu.make_async_copy(col_h.at[pl.ds(tbase, _AG_CH)], cb[bs],
                              semi[bs]).wait()
        pltpu.make_async_copy(w_h.at[pl.ds(tbase, _AG_CH)], wb[bs],
                              semi[bs]).wait()

    def gather_start(bs):
        for g in range(_AG_CH // 16):
            sl = pl.ds(g * 16, 16)
            gb[bs][sl] = rb[bs][sl] + coff
        pltpu.async_copy(hf_h.at[gb[bs]], st[bs], semg[bs])

    def gather_wait(bs):
        pltpu.make_async_copy(hf_h.at[gb[bs]], st[bs], semg[bs]).wait()

    def scale_scatter(bs):
        stg = st[bs]
        for g4 in range(_AG_CH // 16):
            wv = wb[bs][pl.ds(g4 * 16, 16)]
            for l in range(16):
                e = g4 * 16 + l
                nb = wv.at[lax.full((16,), l, _i32)].get(
                    mode="promise_in_bounds")
                for q in range(4):
                    sl2 = pl.ds(q * 16, 16)
                    stg[e, sl2] = stg[e, sl2] * nb
        pltpu.sync_copy(stg, acc.at[cb[bs]], add=True)

    start_idx(0, 0)
    wait_idx(0)
    gather_start(0)
    start_idx(1, 1)

    @pl.loop(0, (_AG_NCH - 1) // 2)
    def _pair(k):
        i2 = k * 2
        wait_idx(1)
        gather_start(1)
        gather_wait(0)
        scale_scatter(0)
        start_idx(i2 + 2, 0)
        wait_idx(0)
        gather_start(0)
        gather_wait(1)
        scale_scatter(1)

        @pl.when(k < (_AG_NCH - 1) // 2 - 1)
        def _pf():
            start_idx(i2 + 3, 1)

    gather_wait(0)
    scale_scatter(0)

    plsc.subcore_barrier()

    obase = pl.multiple_of(coff + rbase, 8)
    for kk in range(11):
        pltpu.sync_copy(acc.at[pl.ds(rbase + kk * 128, 128)], zbuf)
        pltpu.sync_copy(zbuf, agg_h.at[pl.ds(obase + kk * 128, 128)])

    @pl.when(s < _NT - 1)
    def _otail():
        pltpu.sync_copy(acc.at[pl.ds(rbase + 1408, 112)],
                        zbuf.at[pl.ds(0, 112)])
        pltpu.sync_copy(zbuf.at[pl.ds(0, 112)],
                        agg_h.at[pl.ds(obase + 1408, 112)])

    @pl.when(s == _NT - 1)
    def _otail_last():
        pltpu.sync_copy(acc.at[pl.ds(rbase + 1408, 48)],
                        zbuf.at[pl.ds(0, 48)])
        pltpu.sync_copy(zbuf.at[pl.ds(0, 48)],
                        agg_h.at[pl.ds(obase + 1408, 48)])


def _sc_agg(row, col, w, hflat):
    k = pl.kernel(
        _agg_body,
        out_type=jax.ShapeDtypeStruct((2 * _N, _HF), _f32),
        mesh=_mesh(),
        compiler_params=pltpu.CompilerParams(use_tc_tiling_on_sc=False),
        scratch_types=[
            pltpu.VMEM((_AG_CH,), _i32),
            pltpu.VMEM((_AG_CH,), _i32),
            pltpu.VMEM((_AG_CH,), _i32),
            pltpu.VMEM((_AG_CH,), _f32),
            pltpu.VMEM((_AG_CH, _HF), _f32),
            pltpu.VMEM((_AG_CH,), _i32),
            pltpu.VMEM((_AG_CH,), _i32),
            pltpu.VMEM((_AG_CH,), _i32),
            pltpu.VMEM((_AG_CH,), _f32),
            pltpu.VMEM((_AG_CH, _HF), _f32),
            pltpu.SemaphoreType.DMA,
            pltpu.SemaphoreType.DMA,
            pltpu.SemaphoreType.DMA,
            pltpu.SemaphoreType.DMA,
            pltpu.VMEM((128, _HF), _f32),
            pltpu.VMEM_SHARED((_N, _HF), _f32),
        ],
    )
    return k(row, col, w, hflat)


_PL_FULL = _N // 128
_PL_TAIL = _N - _PL_FULL * 128


def _pool_body(xf_h, batch_h, sums_h, cnt_h,
               bbuf, btail, stage, onesv, zbv, zcv, sumS, cntS):
    ci = lax.axis_index("c")
    s = lax.axis_index("s")
    coff = ci * _N

    @pl.loop(0, 64)
    def _z(i):
        for q in range(4):
            zbv[i, pl.ds(q * 16, 16)] = jnp.zeros((16,), _f32)

    @pl.loop(0, 128)
    def _o(i):
        onesv[i, pl.ds(0, 16)] = jnp.ones((16,), _f32)

    @pl.loop(0, 64)
    def _zc(i):
        zcv[i, pl.ds(0, 16)] = jnp.zeros((16,), _f32)

    @pl.when(s == 0)
    def _zero():
        pltpu.sync_copy(zbv, sumS)

    @pl.when(jnp.logical_and(s == 0, ci == 0))
    def _zeroc():
        pltpu.sync_copy(zcv, cntS)

    plsc.subcore_barrier()

    nch = (204 - s) // 16

    @pl.loop(0, nch)
    def _c(k):
        j = s + k * _NT
        base = j * 128
        pltpu.sync_copy(batch_h.at[pl.ds(base, 128)], bbuf)
        pltpu.sync_copy(xf_h.at[pl.ds(coff + base, 128)], stage)
        pltpu.sync_copy(stage, sumS.at[bbuf], add=True)

        @pl.when(ci == 0)
        def _cnt():
            pltpu.sync_copy(onesv, cntS.at[bbuf], add=True)

    @pl.when(s == _NT - 1)
    def _tail():
        base = _PL_FULL * 128
        pltpu.sync_copy(batch_h.at[pl.ds(base, _PL_TAIL)], btail)
        pltpu.sync_copy(xf_h.at[pl.ds(coff + base, _PL_TAIL)],
                        stage.at[pl.ds(0, _PL_TAIL)])
        pltpu.sync_copy(stage.at[pl.ds(0, _PL_TAIL)],
                        sumS.at[btail], add=True)

        @pl.when(ci == 0)
        def _cntt():
            pltpu.sync_copy(onesv.at[pl.ds(0, _PL_TAIL)],
                            cntS.at[btail], add=True)

    plsc.subcore_barrier()

    @pl.when(s == 0)
    def _out():
        pltpu.sync_copy(sumS, zbv)
        pltpu.sync_copy(zbv, sums_h.at[ci])

    @pl.when(jnp.logical_and(s == 0, ci == 0))
    def _outc():
        pltpu.sync_copy(cntS, zcv)
        pltpu.sync_copy(zcv, cnt_h)


def _sc_pool(xflat, batch):
    k = pl.kernel(
        _pool_body,
        out_type=[
            jax.ShapeDtypeStruct((2, _B, _HF), _f32),
            jax.ShapeDtypeStruct((_B, 16), _f32),
        ],
        mesh=_mesh(),
        compiler_params=pltpu.CompilerParams(use_tc_tiling_on_sc=False),
        scratch_types=[
            pltpu.VMEM((128,), _i32),
            pltpu.VMEM((_PL_TAIL,), _i32),
            pltpu.VMEM((128, _HF), _f32),
            pltpu.VMEM((128, 16), _f32),
            pltpu.VMEM((_B, _HF), _f32),
            pltpu.VMEM((_B, 16), _f32),
            pltpu.VMEM_SHARED((_B, _HF), _f32),
            pltpu.VMEM_SHARED((_B, 16), _f32),
        ],
    )
    return k(xflat, batch)


def kernel(x, edge_index, edge_weight, batch, W1, b1, W2, b2, We, be, Wf, bf, lew):
    del edge_weight
    row = edge_index[0]
    col = edge_index[1]

    lew_pad = jnp.pad(lew, ((0, _RP - _R), (0, _RP - _R)))
    tflat = _tc_table(lew_pad).reshape(-1)

    w, degpf = _sc_wdeg(row, col, tflat)
    dinv8 = _tc_deg(degpf.reshape(16, _BN))
    dvcol = dinv8.reshape(-1, 1)

    hs1, xe = _tc_mm(x, W1, We, be.reshape(1, _D), dvcol)
    agg1 = _sc_agg(row, col, w, hs1.reshape(2 * _N, _HF))
    x1, hs2 = _tc_l1(agg1.reshape(2, _N, _HF), hs1, xe, dvcol,
                     b1.reshape(1, _D), W2)
    agg2 = _sc_agg(row, col, w, hs2.reshape(2 * _N, _HF))
    x2 = _tc_l2(agg2.reshape(2, _N, _HF), hs2, x1, dvcol, b2.reshape(1, _D))

    sums, cnt16 = _sc_pool(x2.reshape(2 * _N, _HF), batch)
    return _tc_pred(sums, cnt16, Wf, bf.reshape(1, 1))
